# Initial kernel scaffold; baseline (speedup 1.0000x reference)
import jax
import jax.numpy as jnp
from jax.experimental import pallas as pl


def kernel(input_ids, word_emb, W2, pos_emb, type_emb, ln_g, ln_b):
    raise NotImplementedError("write your pallas kernel here")


# speedup vs baseline: 3.1630x; 3.1630x over previous
import functools

import jax
import jax.numpy as jnp
from jax import lax
from jax.experimental import pallas as pl
from jax.experimental.pallas import tpu as pltpu
from jax.experimental.pallas import tpu_sc as plsc

_V = 100000
_E = 128
_H = 768
_PAD = 1
_B, _S = 4, 2048
_EPS = 1e-12

_NC = 2
_NS = 16
_NW = _NC * _NS
_TOK = _B * _S
_TPW = _TOK // _NW
_CH = 128


def _sc_body(ids_hbm, word_hbm, pos_hbm, we_out, pe_out,
             ids_row, idx_v, pos_v, we_v, pe_v, sem):
    wid = lax.axis_index("s") * _NC + lax.axis_index("c")
    base = wid * _TPW
    b = base // _S
    s0 = base % _S

    pltpu.sync_copy(ids_hbm.at[b], ids_row)

    def _count(j, acc):
        chunk = ids_row[pl.ds(j * 16, 16)]
        is_pad = jnp.where(chunk == _PAD, 1, 0).astype(jnp.int32)
        cnt = jnp.sum(is_pad)
        in_range = jnp.where(j * 16 < s0, 1, 0).astype(jnp.int32)
        return acc + cnt * in_range

    pads_before = lax.fori_loop(0, _S // 16, _count, jnp.int32(0))
    carry = s0 - pads_before

    for cidx in range(_TPW // 16):
        chunk = ids_row[pl.ds(s0 + cidx * 16, 16)]
        m = jnp.where(chunk != _PAD, 1, 0).astype(jnp.int32)
        cum = plsc.cumsum(m) + carry
        posv = cum * m + _PAD
        r, c0 = cidx // (_CH // 16), (cidx % (_CH // 16)) * 16
        idx_v[r, pl.ds(c0, 16)] = chunk
        pos_v[r, pl.ds(c0, 16)] = posv
        carry = carry + jnp.sum(m)

    for j in range(_TPW // _CH):
        pltpu.async_copy(word_hbm.at[idx_v.at[j]], we_v, sem).wait()
        pltpu.sync_copy(we_v, we_out.at[pl.ds(base + j * _CH, _CH)])
    for j in range(_TPW // _CH):
        pltpu.async_copy(pos_hbm.at[pos_v.at[j]], pe_v, sem).wait()
        pltpu.sync_copy(pe_v, pe_out.at[pl.ds(base + j * _CH, _CH)])


@functools.cache
def _sc_gather():
    return pl.kernel(
        _sc_body,
        out_type=[
            jax.ShapeDtypeStruct((_TOK, _E), jnp.float32),
            jax.ShapeDtypeStruct((_TOK, _H), jnp.float32),
        ],
        mesh=plsc.VectorSubcoreMesh(core_axis_name="c", subcore_axis_name="s"),
        compiler_params=pltpu.CompilerParams(needs_layout_passes=False),
        scratch_types=[
            pltpu.VMEM((_S,), jnp.int32),
            pltpu.VMEM((_TPW // _CH, _CH), jnp.int32),
            pltpu.VMEM((_TPW // _CH, _CH), jnp.int32),
            pltpu.VMEM((_CH, _E), jnp.float32),
            pltpu.VMEM((_CH, _H), jnp.float32),
            pltpu.SemaphoreType.DMA,
        ],
    )


_BLK = 512


def _tc_body(we_ref, w2_ref, pe_ref, te_ref, g_ref, b_ref, o_ref):
    x = jnp.dot(we_ref[...], w2_ref[...], preferred_element_type=jnp.float32)
    x = x + pe_ref[...] + te_ref[...]
    mu = jnp.mean(x, axis=-1, keepdims=True)
    d = x - mu
    var = jnp.mean(d * d, axis=-1, keepdims=True)
    o_ref[...] = d * lax.rsqrt(var + _EPS) * g_ref[...] + b_ref[...]


def _tc_call(we, w2, pe, te, g, b):
    return pl.pallas_call(
        _tc_body,
        grid=(_TOK // _BLK,),
        in_specs=[
            pl.BlockSpec((_BLK, _E), lambda i: (i, 0)),
            pl.BlockSpec((_E, _H), lambda i: (0, 0)),
            pl.BlockSpec((_BLK, _H), lambda i: (i, 0)),
            pl.BlockSpec((1, _H), lambda i: (0, 0)),
            pl.BlockSpec((1, _H), lambda i: (0, 0)),
            pl.BlockSpec((1, _H), lambda i: (0, 0)),
        ],
        out_specs=pl.BlockSpec((_BLK, _H), lambda i: (i, 0)),
        out_shape=jax.ShapeDtypeStruct((_TOK, _H), jnp.float32),
    )(we, w2, pe, te, g, b)


def kernel(input_ids, word_emb, W2, pos_emb, type_emb, ln_g, ln_b):
    ids = input_ids.astype(jnp.int32)
    we, pe = _sc_gather()(ids, word_emb, pos_emb)
    out = _tc_call(we, W2, pe, type_emb[0:1],
                   ln_g.reshape(1, _H), ln_b.reshape(1, _H))
    return out.reshape(_B, _S, _H)


# speedup vs baseline: 3.4385x; 1.0871x over previous
import functools

import jax
import jax.numpy as jnp
from jax import lax
from jax.experimental import pallas as pl
from jax.experimental.pallas import tpu as pltpu
from jax.experimental.pallas import tpu_sc as plsc

_V = 100000
_E = 128
_H = 768
_PAD = 1
_B, _S = 4, 2048
_EPS = 1e-12

_NC = 2
_NS = 16
_NW = _NC * _NS
_TOK = _B * _S

_NCHUNK = 2
_ROWS_PER_CHUNK = _B // _NCHUNK
_CTOK = _TOK // _NCHUNK
_TPW = _CTOK // _NW


def _sc_body(ids_hbm, word_hbm, pos_hbm, we_out, pe_out,
             ids_row, idx_v, pos_v, we_v, pe_v, sem_w, sem_p):
    wid = lax.axis_index("s") * _NC + lax.axis_index("c")
    base = wid * _TPW
    b = base // _S
    s0 = base % _S

    pltpu.sync_copy(ids_hbm.at[b, pl.ds(s0, _TPW)], idx_v)
    word_cp = pltpu.async_copy(word_hbm.at[idx_v], we_v, sem_w)

    pltpu.sync_copy(ids_hbm.at[b], ids_row)

    def _count(j, acc):
        chunk = ids_row[pl.ds(j * 16, 16)]
        is_pad = jnp.where(chunk == _PAD, 1, 0).astype(jnp.int32)
        cnt = jnp.sum(is_pad)
        in_range = jnp.where(j * 16 < s0, 1, 0).astype(jnp.int32)
        return acc + cnt * in_range

    pads_before = lax.fori_loop(0, _S // 16, _count, jnp.int32(0))
    carry = s0 - pads_before

    for cidx in range(_TPW // 16):
        chunk = ids_row[pl.ds(s0 + cidx * 16, 16)]
        m = jnp.where(chunk != _PAD, 1, 0).astype(jnp.int32)
        cum = plsc.cumsum(m) + carry
        pos_v[pl.ds(cidx * 16, 16)] = cum * m + _PAD
        carry = carry + jnp.sum(m)

    pos_cp = pltpu.async_copy(pos_hbm.at[pos_v], pe_v, sem_p)
    word_cp.wait()
    pltpu.sync_copy(we_v, we_out.at[pl.ds(base, _TPW)])
    pos_cp.wait()
    pltpu.sync_copy(pe_v, pe_out.at[pl.ds(base, _TPW)])


@functools.cache
def _sc_gather():
    return pl.kernel(
        _sc_body,
        out_type=[
            jax.ShapeDtypeStruct((_CTOK, _E), jnp.float32),
            jax.ShapeDtypeStruct((_CTOK, _H), jnp.float32),
        ],
        mesh=plsc.VectorSubcoreMesh(core_axis_name="c", subcore_axis_name="s"),
        compiler_params=pltpu.CompilerParams(needs_layout_passes=False),
        scratch_types=[
            pltpu.VMEM((_S,), jnp.int32),
            pltpu.VMEM((_TPW,), jnp.int32),
            pltpu.VMEM((_TPW,), jnp.int32),
            pltpu.VMEM((_TPW, _E), jnp.float32),
            pltpu.VMEM((_TPW, _H), jnp.float32),
            pltpu.SemaphoreType.DMA,
            pltpu.SemaphoreType.DMA,
        ],
    )


_BLK = 512
_STEPS_PER_CHUNK = _CTOK // _BLK


def _tc_body(we_ref, w2_ref, pe_ref, te_ref, g_ref, b_ref, o_ref):
    x = jnp.dot(we_ref[...], w2_ref[...], preferred_element_type=jnp.float32)
    x = x + pe_ref[...] + te_ref[...]
    mu = jnp.mean(x, axis=-1, keepdims=True)
    d = x - mu
    var = jnp.mean(d * d, axis=-1, keepdims=True)
    o_ref[...] = d * lax.rsqrt(var + _EPS) * g_ref[...] + b_ref[...]


@functools.cache
def _tc_call(chunk):
    base_blk = chunk * _STEPS_PER_CHUNK
    body = _tc_body if chunk == 0 else (lambda buf, *refs: _tc_body(*refs))
    specs = [
        pl.BlockSpec((_BLK, _E), lambda i: (i, 0)),
        pl.BlockSpec((_E, _H), lambda i: (0, 0)),
        pl.BlockSpec((_BLK, _H), lambda i: (i, 0)),
        pl.BlockSpec((1, _H), lambda i: (0, 0)),
        pl.BlockSpec((1, _H), lambda i: (0, 0)),
        pl.BlockSpec((1, _H), lambda i: (0, 0)),
    ]
    if chunk > 0:
        specs = [pl.BlockSpec(memory_space=pl.ANY)] + specs
    return pl.pallas_call(
        body,
        grid=(_STEPS_PER_CHUNK,),
        in_specs=specs,
        out_specs=pl.BlockSpec((_BLK, _H), lambda i: (base_blk + i, 0)),
        out_shape=jax.ShapeDtypeStruct((_TOK, _H), jnp.float32),
        input_output_aliases={0: 0} if chunk > 0 else {},
    )


def kernel(input_ids, word_emb, W2, pos_emb, type_emb, ln_g, ln_b):
    ids = input_ids.astype(jnp.int32)
    te = type_emb[0:1]
    g = ln_g.reshape(1, _H)
    b = ln_b.reshape(1, _H)

    sc = _sc_gather()
    gathered = [sc(ids[c * _ROWS_PER_CHUNK:(c + 1) * _ROWS_PER_CHUNK],
                   word_emb, pos_emb)
                for c in range(_NCHUNK)]

    buf = None
    for c in range(_NCHUNK):
        we, pe = gathered[c]
        if c == 0:
            buf = _tc_call(c)(we, W2, pe, te, g, b)
        else:
            buf = _tc_call(c)(buf, we, W2, pe, te, g, b)
    return buf.reshape(_B, _S, _H)


# speedup vs baseline: 3.5548x; 1.0338x over previous
import functools

import jax
import jax.numpy as jnp
from jax import lax
from jax.experimental import pallas as pl
from jax.experimental.pallas import tpu as pltpu
from jax.experimental.pallas import tpu_sc as plsc

_V = 100000
_E = 128
_H = 768
_PAD = 1
_B, _S = 4, 2048
_EPS = 1e-12

_NC = 2
_NS = 16
_NW = _NC * _NS
_TOK = _B * _S
_TPW = _TOK // _NW
_CH = 128

_BLK = 512
_NBLK = _TOK // _BLK
_BPR = _S // _BLK
_WIN = 2176
_WCOLS = 640


def _sc_body(ids_hbm, word_hbm, we_out, idx_v, we_v, sem):
    wid = lax.axis_index("s") * _NC + lax.axis_index("c")
    base = wid * _TPW
    for j in range(_TPW // _CH):
        pltpu.sync_copy(ids_hbm.at[pl.ds(base + j * _CH, _CH)], idx_v.at[j])
        pltpu.async_copy(word_hbm.at[idx_v.at[j]], we_v, sem).wait()
        pltpu.sync_copy(we_v, we_out.at[pl.ds(base + j * _CH, _CH)])


@functools.cache
def _sc_gather():
    return pl.kernel(
        _sc_body,
        out_type=jax.ShapeDtypeStruct((_TOK, _E), jnp.float32),
        mesh=plsc.VectorSubcoreMesh(core_axis_name="c", subcore_axis_name="s"),
        compiler_params=pltpu.CompilerParams(needs_layout_passes=False),
        scratch_types=[
            pltpu.VMEM((_TPW // _CH, _CH), jnp.int32),
            pltpu.VMEM((_CH, _E), jnp.float32),
            pltpu.SemaphoreType.DMA,
        ],
    )


def _tc_body(we_ref, w2_ref, win_ref, ids_ref, tril_ref, te_ref, pad_ref,
             g_ref, b_ref, o_ref, c_ref):
    i = pl.program_id(0)

    c = jnp.where(i % _BPR == 0, 0.0, c_ref[0])

    ids = ids_ref[...]
    m = jnp.where(ids != _PAD, 1.0, 0.0)

    cum = jnp.dot(tril_ref[...], m.astype(jnp.bfloat16),
                  preferred_element_type=jnp.float32)

    pos = (cum + c) * m + float(_PAD)
    ci = c.astype(jnp.int32)
    q0 = ((ci + 2) // 8) * 8

    col = pos - q0.astype(jnp.float32)
    iota = lax.broadcasted_iota(jnp.int32, (_BLK, _WCOLS), 1).astype(jnp.float32)
    onehot = jnp.where((col == iota) & (m > 0.0), 1.0, 0.0).astype(jnp.bfloat16)

    win = win_ref[pl.ds(q0, _WCOLS), :].astype(jnp.bfloat16)
    pe = jnp.dot(onehot, win, preferred_element_type=jnp.float32)
    pe = pe + (1.0 - m) * pad_ref[...]

    x = jnp.dot(we_ref[...], w2_ref[...], preferred_element_type=jnp.float32)
    x = x + te_ref[...] + pe
    mu = jnp.mean(x, axis=-1, keepdims=True)
    d = x - mu
    var = jnp.mean(d * d, axis=-1, keepdims=True)
    o_ref[...] = d * lax.rsqrt(var + _EPS) * g_ref[...] + b_ref[...]

    c_ref[0] = c + jnp.sum(m)


@functools.cache
def _tc_call(interpret=False):
    return pl.pallas_call(
        _tc_body,
        grid=(_NBLK,),
        in_specs=[
            pl.BlockSpec((_BLK, _E), lambda i: (i, 0)),
            pl.BlockSpec((_E, _H), lambda i: (0, 0)),
            pl.BlockSpec((_WIN, _H), lambda i: (0, 0)),
            pl.BlockSpec((_BLK, 1), lambda i: (i, 0)),
            pl.BlockSpec((_BLK, _BLK), lambda i: (0, 0)),
            pl.BlockSpec((1, _H), lambda i: (0, 0)),
            pl.BlockSpec((1, _H), lambda i: (0, 0)),
            pl.BlockSpec((1, _H), lambda i: (0, 0)),
            pl.BlockSpec((1, _H), lambda i: (0, 0)),
        ],
        out_specs=pl.BlockSpec((_BLK, _H), lambda i: (i, 0)),
        out_shape=jax.ShapeDtypeStruct((_TOK, _H), jnp.float32),
        scratch_shapes=[pltpu.SMEM((1,), jnp.float32)],
        interpret=interpret,
    )


def kernel(input_ids, word_emb, W2, pos_emb, type_emb, ln_g, ln_b):
    ids = input_ids.astype(jnp.int32).reshape(_TOK)
    we = _sc_gather()(ids, word_emb)
    tril = (jnp.arange(_BLK)[:, None] >= jnp.arange(_BLK)[None, :]
            ).astype(jnp.bfloat16)
    out = _tc_call()(
        we, W2, pos_emb[:_WIN], ids.reshape(_TOK, 1), tril, type_emb[0:1],
        pos_emb[_PAD:_PAD + 1], ln_g.reshape(1, _H), ln_b.reshape(1, _H))
    return out.reshape(_B, _S, _H)


# speedup vs baseline: 3.8023x; 1.0696x over previous
import functools

import jax
import jax.numpy as jnp
from jax import lax
from jax.experimental import pallas as pl
from jax.experimental.pallas import tpu as pltpu
from jax.experimental.pallas import tpu_sc as plsc

_V = 100000
_E = 128
_H = 768
_PAD = 1
_B, _S = 4, 2048
_EPS = 1e-12

_NC = 2
_NS = 16
_NW = _NC * _NS
_TOK = _B * _S
_TPW = _TOK // _NW
_CH = 128

_BLK = 512
_NBLK = _TOK // _BLK
_BPR = _S // _BLK
_WIN = 2176
_WCOLS = 640


def _sc_body(ids_hbm, word_hbm, we_out, idx_v, we_v, sem):
    wid = lax.axis_index("s") * _NC + lax.axis_index("c")
    base = wid * _TPW
    for j in range(_TPW // _CH):
        pltpu.sync_copy(ids_hbm.at[pl.ds(base + j * _CH, _CH)], idx_v.at[j])
        pltpu.async_copy(word_hbm.at[idx_v.at[j]], we_v, sem).wait()
        pltpu.sync_copy(we_v, we_out.at[pl.ds(base + j * _CH, _CH)])


@functools.cache
def _sc_gather():
    return pl.kernel(
        _sc_body,
        out_type=jax.ShapeDtypeStruct((_TOK, _E), jnp.float32),
        mesh=plsc.VectorSubcoreMesh(core_axis_name="c", subcore_axis_name="s"),
        compiler_params=pltpu.CompilerParams(needs_layout_passes=False),
        scratch_types=[
            pltpu.VMEM((_TPW // _CH, _CH), jnp.int32),
            pltpu.VMEM((_CH, _E), jnp.float32),
            pltpu.SemaphoreType.DMA,
        ],
    )


def _tc_body(we_ref, w2_ref, win_ref, ids_ref, tril_ref, te_ref, pad_ref,
             g_ref, b_ref, o_ref, c_ref):
    i = pl.program_id(0)

    c = jnp.where(i % _BPR == 0, 0.0, c_ref[0])

    ids = ids_ref[...]
    m = jnp.where(ids != _PAD, 1.0, 0.0)

    cum = jnp.dot(tril_ref[...], m.astype(jnp.bfloat16),
                  preferred_element_type=jnp.float32)

    pos = (cum + c) * m + float(_PAD)
    ci = c.astype(jnp.int32)
    q0 = ((ci + 2) // 8) * 8

    col = pos - q0.astype(jnp.float32)
    iota = lax.broadcasted_iota(jnp.int32, (_BLK, _WCOLS), 1).astype(jnp.float32)
    onehot = jnp.where((col == iota) & (m > 0.0), 1.0, 0.0).astype(jnp.bfloat16)

    win = win_ref[pl.ds(q0, _WCOLS), :].astype(jnp.bfloat16)
    pe = jnp.dot(onehot, win, preferred_element_type=jnp.float32)
    pe = pe + (1.0 - m) * pad_ref[...]

    x = jnp.dot(we_ref[...].astype(jnp.bfloat16), w2_ref[...],
                preferred_element_type=jnp.float32)
    x = x + te_ref[...] + pe
    mu = jnp.mean(x, axis=-1, keepdims=True)
    d = x - mu
    var = jnp.mean(d * d, axis=-1, keepdims=True)
    o_ref[...] = d * lax.rsqrt(var + _EPS) * g_ref[...] + b_ref[...]

    c_ref[0] = c + jnp.sum(m)


@functools.cache
def _tc_call(interpret=False):
    return pl.pallas_call(
        _tc_body,
        grid=(_NBLK,),
        in_specs=[
            pl.BlockSpec((_BLK, _E), lambda i: (i, 0)),
            pl.BlockSpec((_E, _H), lambda i: (0, 0)),
            pl.BlockSpec((_WIN, _H), lambda i: (0, 0)),
            pl.BlockSpec((_BLK, 1), lambda i: (i, 0)),
            pl.BlockSpec((_BLK, _BLK), lambda i: (0, 0)),
            pl.BlockSpec((1, _H), lambda i: (0, 0)),
            pl.BlockSpec((1, _H), lambda i: (0, 0)),
            pl.BlockSpec((1, _H), lambda i: (0, 0)),
            pl.BlockSpec((1, _H), lambda i: (0, 0)),
        ],
        out_specs=pl.BlockSpec((_BLK, _H), lambda i: (i, 0)),
        out_shape=jax.ShapeDtypeStruct((_TOK, _H), jnp.float32),
        scratch_shapes=[pltpu.SMEM((1,), jnp.float32)],
        interpret=interpret,
    )


def kernel(input_ids, word_emb, W2, pos_emb, type_emb, ln_g, ln_b):
    ids = input_ids.astype(jnp.int32).reshape(_TOK)
    we = _sc_gather()(ids, word_emb)
    tril = (jnp.arange(_BLK)[:, None] >= jnp.arange(_BLK)[None, :]
            ).astype(jnp.bfloat16)
    out = _tc_call()(
        we, W2.astype(jnp.bfloat16), pos_emb, ids.reshape(_TOK, 1), tril,
        type_emb[0:1], pos_emb[_PAD:_PAD + 1], ln_g.reshape(1, _H),
        ln_b.reshape(1, _H))
    return out.reshape(_B, _S, _H)


# speedup vs baseline: 4.0524x; 1.0658x over previous
import functools

import jax
import jax.numpy as jnp
from jax import lax
from jax.experimental import pallas as pl
from jax.experimental.pallas import tpu as pltpu
from jax.experimental.pallas import tpu_sc as plsc

_V = 100000
_E = 128
_H = 768
_PAD = 1
_B, _S = 4, 2048
_EPS = 1e-12

_NC = 2
_NS = 16
_NW = _NC * _NS
_TOK = _B * _S
_TPW = _TOK // _NW
_CH = 128

_BLK = 512
_NBLK = _TOK // _BLK
_BPR = _S // _BLK
_WIN = 2176
_HB = 256
_WCOLS = 384


def _sc_body(ids_hbm, word_hbm, we_out, idx_v, we_v, sem):
    wid = lax.axis_index("s") * _NC + lax.axis_index("c")
    base = wid * _TPW
    for j in range(_TPW // _CH):
        pltpu.sync_copy(ids_hbm.at[pl.ds(base + j * _CH, _CH)], idx_v.at[j])
        pltpu.async_copy(word_hbm.at[idx_v.at[j]], we_v, sem).wait()
        pltpu.sync_copy(we_v, we_out.at[pl.ds(base + j * _CH, _CH)])


@functools.cache
def _sc_gather():
    return pl.kernel(
        _sc_body,
        out_type=jax.ShapeDtypeStruct((_TOK, _E), jnp.float32),
        mesh=plsc.VectorSubcoreMesh(core_axis_name="c", subcore_axis_name="s"),
        compiler_params=pltpu.CompilerParams(needs_layout_passes=False),
        scratch_types=[
            pltpu.VMEM((_TPW // _CH, _CH), jnp.int32),
            pltpu.VMEM((_CH, _E), jnp.float32),
            pltpu.SemaphoreType.DMA,
        ],
    )


def _tc_body(we_ref, w2_ref, win_ref, ids_ref, tril_ref, te_ref, pad_ref,
             g_ref, b_ref, o_ref, c_ref):
    i = pl.program_id(0)

    c = jnp.where(i % _BPR == 0, 0.0, c_ref[0])

    ids = ids_ref[...]
    m = jnp.where(ids != _PAD, 1.0, 0.0)

    cum = jnp.dot(tril_ref[...], m.astype(jnp.bfloat16),
                  preferred_element_type=jnp.float32)

    pos = (cum + c) * m + float(_PAD)

    iota = lax.broadcasted_iota(jnp.int32, (_HB, _WCOLS), 1).astype(jnp.float32)
    halves = []
    for h in range(2):
        ch = c if h == 0 else c + jnp.sum(m[:_HB, :])
        q0 = ((ch.astype(jnp.int32) + 2) // 8) * 8
        ph = pos[h * _HB:(h + 1) * _HB, :]
        mh = m[h * _HB:(h + 1) * _HB, :]
        col = ph - q0.astype(jnp.float32)
        onehot = jnp.where((col == iota) & (mh > 0.0), 1.0, 0.0
                           ).astype(jnp.bfloat16)
        win = win_ref[pl.ds(q0, _WCOLS), :].astype(jnp.bfloat16)
        halves.append(jnp.dot(onehot, win, preferred_element_type=jnp.float32))
    pe = jnp.concatenate(halves, axis=0)
    pe = pe + (1.0 - m) * pad_ref[...]

    x = jnp.dot(we_ref[...].astype(jnp.bfloat16), w2_ref[...],
                preferred_element_type=jnp.float32)
    x = x + te_ref[...] + pe
    mu = jnp.mean(x, axis=-1, keepdims=True)
    d = x - mu
    var = jnp.mean(d * d, axis=-1, keepdims=True)
    o_ref[...] = d * lax.rsqrt(var + _EPS) * g_ref[...] + b_ref[...]

    c_ref[0] = c + jnp.sum(m)


@functools.cache
def _tc_call(interpret=False):
    return pl.pallas_call(
        _tc_body,
        grid=(_NBLK,),
        in_specs=[
            pl.BlockSpec((_BLK, _E), lambda i: (i, 0)),
            pl.BlockSpec((_E, _H), lambda i: (0, 0)),
            pl.BlockSpec((_WIN, _H), lambda i: (0, 0)),
            pl.BlockSpec((_BLK, 1), lambda i: (i, 0)),
            pl.BlockSpec((_BLK, _BLK), lambda i: (0, 0)),
            pl.BlockSpec((1, _H), lambda i: (0, 0)),
            pl.BlockSpec((1, _H), lambda i: (0, 0)),
            pl.BlockSpec((1, _H), lambda i: (0, 0)),
            pl.BlockSpec((1, _H), lambda i: (0, 0)),
        ],
        out_specs=pl.BlockSpec((_BLK, _H), lambda i: (i, 0)),
        out_shape=jax.ShapeDtypeStruct((_TOK, _H), jnp.float32),
        scratch_shapes=[pltpu.SMEM((1,), jnp.float32)],
        interpret=interpret,
    )


def kernel(input_ids, word_emb, W2, pos_emb, type_emb, ln_g, ln_b):
    ids = input_ids.astype(jnp.int32).reshape(_TOK)
    we = _sc_gather()(ids, word_emb)
    tril = (jnp.arange(_BLK)[:, None] >= jnp.arange(_BLK)[None, :]
            ).astype(jnp.bfloat16)
    out = _tc_call()(
        we, W2.astype(jnp.bfloat16), pos_emb, ids.reshape(_TOK, 1), tril,
        type_emb[0:1], pos_emb[_PAD:_PAD + 1], ln_g.reshape(1, _H),
        ln_b.reshape(1, _H))
    return out.reshape(_B, _S, _H)
